# initial kernel scaffold (unmeasured)
import jax
import jax.numpy as jnp
from jax import lax
from jax.experimental import pallas as pl
from jax.experimental.pallas import tpu as pltpu

N_Z = 4


def kernel(O, Wo):
    B, S, Hs, D = O.shape
    K = Hs * D
    N = Wo.shape[1]
    S_out = S // N_Z

    O2 = O.reshape(B, S, K)

    def body(o_ref, w_ref, out_ref, comm_ref, send_sems, recv_sems, credit_sem):
        my_x = lax.axis_index("x")
        my_y = lax.axis_index("y")
        my_z = lax.axis_index("z")
        left = (my_z - 1) % N_Z
        right = (my_z + 1) % N_Z

        barrier_sem = pltpu.get_barrier_semaphore()
        for nbr in (left, right):
            pl.semaphore_signal(
                barrier_sem, inc=1,
                device_id=(my_x, my_y, nbr),
                device_id_type=pl.DeviceIdType.MESH,
            )
        pl.semaphore_wait(barrier_sem, 2)

        def partial(c, b):
            return jnp.dot(
                o_ref[b, pl.ds(c * S_out, S_out), :],
                w_ref[...],
                preferred_element_type=jnp.float32,
            )

        for b in range(B):
            comm_ref[0, b, :, :] = partial((my_z - 1) % N_Z, b)

        for t in range(N_Z - 1):
            s_slot = t
            r_slot = (t + 1) % 3
            if t == N_Z - 2:
                pl.semaphore_wait(credit_sem, 1)
            rdma = pltpu.make_async_remote_copy(
                src_ref=comm_ref.at[s_slot],
                dst_ref=comm_ref.at[r_slot],
                send_sem=send_sems.at[t],
                recv_sem=recv_sems.at[t],
                device_id=(my_x, my_y, right),
                device_id_type=pl.DeviceIdType.MESH,
            )
            rdma.start()
            rdma.wait()
            if t == 0:
                pl.semaphore_signal(
                    credit_sem, inc=1,
                    device_id=(my_x, my_y, left),
                    device_id_type=pl.DeviceIdType.MESH,
                )
            c = (my_z - 2 - t) % N_Z
            if t < N_Z - 2:
                for b in range(B):
                    comm_ref[r_slot, b, :, :] = (
                        comm_ref[r_slot, b, :, :] + partial(c, b)
                    )
            else:
                for b in range(B):
                    out_ref[b, :, :] = comm_ref[r_slot, b, :, :] + partial(c, b)

    return pl.pallas_call(
        body,
        out_shape=jax.ShapeDtypeStruct((B, S_out, N), jnp.float32),
        in_specs=[
            pl.BlockSpec(memory_space=pltpu.VMEM),
            pl.BlockSpec(memory_space=pltpu.VMEM),
        ],
        out_specs=pl.BlockSpec(memory_space=pltpu.VMEM),
        scratch_shapes=[
            pltpu.VMEM((3, B, S_out, N), jnp.float32),
            pltpu.SemaphoreType.DMA((N_Z - 1,)),
            pltpu.SemaphoreType.DMA((N_Z - 1,)),
            pltpu.SemaphoreType.REGULAR,
        ],
        compiler_params=pltpu.CompilerParams(collective_id=0),
    )(O2, Wo)


# baseline (device time: 630372 ns/iter reference)
import jax
import jax.numpy as jnp
from jax import lax
from jax.experimental import pallas as pl
from jax.experimental.pallas import tpu as pltpu

N_Z = 4


def kernel(O, Wo):
    B, S, Hs, D = O.shape
    K = Hs * D
    N = Wo.shape[1]
    S_out = S // N_Z

    O2 = O.reshape(B, S, K)

    def body(o_hbm, w_hbm, out_hbm, w_vmem, o_stage, comm_ref,
             send_sems, recv_sems, local_sems, credit_sem):
        my_x = lax.axis_index("x")
        my_y = lax.axis_index("y")
        my_z = lax.axis_index("z")
        left = (my_z - 1) % N_Z
        right = (my_z + 1) % N_Z

        barrier_sem = pltpu.get_barrier_semaphore()
        for nbr in (left, right):
            pl.semaphore_signal(
                barrier_sem, inc=1,
                device_id=(my_x, my_y, nbr),
                device_id_type=pl.DeviceIdType.MESH,
            )
        pl.semaphore_wait(barrier_sem, 2)

        def stage_o(c, buf):
            cp = pltpu.make_async_copy(
                o_hbm.at[:, pl.ds(c * S_out, S_out), :],
                o_stage.at[buf],
                local_sems.at[1],
            )
            cp.start()
            return cp

        def accum(slot, buf, init):
            for b in range(B):
                for h in range(2):
                    cols = slice(h * (N // 2), (h + 1) * (N // 2))
                    p = jnp.dot(
                        o_stage[buf, b, :, :],
                        w_vmem[:, cols],
                        preferred_element_type=jnp.float32,
                    )
                    if init:
                        comm_ref[slot, b, :, cols] = p
                    else:
                        comm_ref[slot, b, :, cols] = (
                            comm_ref[slot, b, :, cols] + p
                        )

        wo_cp = pltpu.make_async_copy(w_hbm, w_vmem, local_sems.at[0])
        wo_cp.start()
        o_cp = stage_o((my_z - 1) % N_Z, 0)
        wo_cp.wait()
        o_cp.wait()

        accum(0, 0, init=True)

        for t in range(N_Z - 1):
            s_slot = t % 2
            r_slot = (t + 1) % 2
            if t >= 1:
                pl.semaphore_wait(credit_sem, 1)
            rdma = pltpu.make_async_remote_copy(
                src_ref=comm_ref.at[s_slot],
                dst_ref=comm_ref.at[r_slot],
                send_sem=send_sems.at[t],
                recv_sem=recv_sems.at[t],
                device_id=(my_x, my_y, right),
                device_id_type=pl.DeviceIdType.MESH,
            )
            rdma.start()
            c_next = (my_z - 2 - t) % N_Z
            o_buf = (t + 1) % 2
            o_cp = stage_o(c_next, o_buf)
            rdma.wait()
            if t <= 1:
                pl.semaphore_signal(
                    credit_sem, inc=1,
                    device_id=(my_x, my_y, left),
                    device_id_type=pl.DeviceIdType.MESH,
                )
            o_cp.wait()
            accum(r_slot, o_buf, init=False)

        out_cp = pltpu.make_async_copy(comm_ref.at[1], out_hbm, local_sems.at[2])
        out_cp.start()
        out_cp.wait()

    return pl.pallas_call(
        body,
        out_shape=jax.ShapeDtypeStruct((B, S_out, N), jnp.float32),
        in_specs=[
            pl.BlockSpec(memory_space=pltpu.MemorySpace.HBM),
            pl.BlockSpec(memory_space=pltpu.MemorySpace.HBM),
        ],
        out_specs=pl.BlockSpec(memory_space=pltpu.MemorySpace.HBM),
        scratch_shapes=[
            pltpu.VMEM((K, N), jnp.float32),
            pltpu.VMEM((2, B, S_out, K), jnp.float32),
            pltpu.VMEM((2, B, S_out, N), jnp.float32),
            pltpu.SemaphoreType.DMA((N_Z - 1,)),
            pltpu.SemaphoreType.DMA((N_Z - 1,)),
            pltpu.SemaphoreType.DMA((3,)),
            pltpu.SemaphoreType.REGULAR,
        ],
        compiler_params=pltpu.CompilerParams(
            collective_id=0,
            vmem_limit_bytes=64 * 1024 * 1024,
        ),
    )(O2, Wo)


# device time: 603744 ns/iter; 1.0441x vs baseline; 1.0441x over previous
import jax
import jax.numpy as jnp
from jax import lax
from jax.experimental import pallas as pl
from jax.experimental.pallas import tpu as pltpu

N_Z = 4
Q = 4


def kernel(O, Wo):
    B, S, Hs, D = O.shape
    K = Hs * D
    N = Wo.shape[1]
    S_out = S // N_Z
    RQ = S_out // Q

    O2 = O.reshape(B, S, K)

    def body(o_hbm, w_hbm, out_hbm, w_vmem, o_stage, comm_ref,
             send_sems, recv_sems, local_sems, credit_sem):
        my_x = lax.axis_index("x")
        my_y = lax.axis_index("y")
        my_z = lax.axis_index("z")
        left = (my_z - 1) % N_Z
        right = (my_z + 1) % N_Z

        barrier_sem = pltpu.get_barrier_semaphore()
        for nbr in (left, right):
            pl.semaphore_signal(
                barrier_sem, inc=1,
                device_id=(my_x, my_y, nbr),
                device_id_type=pl.DeviceIdType.MESH,
            )
        pl.semaphore_wait(barrier_sem, 2)

        def stage_o(c, buf):
            cp = pltpu.make_async_copy(
                o_hbm.at[:, pl.ds(c * S_out, S_out), :],
                o_stage.at[buf],
                local_sems.at[1 + buf],
            )
            cp.start()
            return cp

        def accum_block(slot, q, buf, init):
            for b in range(B):
                p = jnp.dot(
                    o_stage[buf, b, q * RQ:(q + 1) * RQ, :],
                    w_vmem[...],
                    preferred_element_type=jnp.float32,
                )
                if init:
                    comm_ref[slot, q, b, :, :] = p
                else:
                    comm_ref[slot, q, b, :, :] = comm_ref[slot, q, b, :, :] + p

        def make_rdma(t, q, s_slot, r_slot):
            return pltpu.make_async_remote_copy(
                src_ref=comm_ref.at[s_slot, q],
                dst_ref=comm_ref.at[r_slot, q],
                send_sem=send_sems.at[t, q],
                recv_sem=recv_sems.at[t, q],
                device_id=(my_x, my_y, right),
                device_id_type=pl.DeviceIdType.MESH,
            )

        wo_cp = pltpu.make_async_copy(w_hbm, w_vmem, local_sems.at[0])
        wo_cp.start()
        o_cp0 = stage_o((my_z - 1) % N_Z, 0)
        wo_cp.wait()
        o_cp0.wait()

        rdmas = {}

        for q in range(Q):
            accum_block(0, q, 0, init=True)
            r = make_rdma(0, q, 0, 1)
            r.start()
            rdmas[(0, q)] = r
        o_cp1 = stage_o((my_z - 2) % N_Z, 1)

        for t in range(1, N_Z - 1):
            s_slot = t % 2
            r_slot = (t + 1) % 2
            o_buf = t % 2
            o_cp = o_cp1 if o_buf == 1 else o_cp0
            o_cp.wait()
            for q in range(Q):
                rdmas[(t - 1, q)].wait_recv()
                accum_block(s_slot, q, o_buf, init=False)
            nxt = stage_o((my_z - 2 - t) % N_Z, (t + 1) % 2)
            if (t + 1) % 2 == 1:
                o_cp1 = nxt
            else:
                o_cp0 = nxt
            for q in range(Q):
                rdmas[(t - 1, q)].wait_send()
            pl.semaphore_signal(
                credit_sem, inc=1,
                device_id=(my_x, my_y, left),
                device_id_type=pl.DeviceIdType.MESH,
            )
            pl.semaphore_wait(credit_sem, 1)
            for q in range(Q):
                r = make_rdma(t, q, s_slot, r_slot)
                r.start()
                rdmas[(t, q)] = r

        o_cp1.wait()
        out_cps = []
        for q in range(Q):
            rdmas[(N_Z - 2, q)].wait_recv()
            accum_block(1, q, 1, init=False)
            for b in range(B):
                cp = pltpu.make_async_copy(
                    comm_ref.at[1, q, b],
                    out_hbm.at[b, pl.ds(q * RQ, RQ), :],
                    local_sems.at[3],
                )
                cp.start()
                out_cps.append(cp)
        for cp in out_cps:
            cp.wait()
        for q in range(Q):
            rdmas[(N_Z - 2, q)].wait_send()

    return pl.pallas_call(
        body,
        out_shape=jax.ShapeDtypeStruct((B, S_out, N), jnp.float32),
        in_specs=[
            pl.BlockSpec(memory_space=pltpu.MemorySpace.HBM),
            pl.BlockSpec(memory_space=pltpu.MemorySpace.HBM),
        ],
        out_specs=pl.BlockSpec(memory_space=pltpu.MemorySpace.HBM),
        scratch_shapes=[
            pltpu.VMEM((K, N), jnp.float32),
            pltpu.VMEM((2, B, S_out, K), jnp.float32),
            pltpu.VMEM((2, Q, B, RQ, N), jnp.float32),
            pltpu.SemaphoreType.DMA((N_Z - 1, Q)),
            pltpu.SemaphoreType.DMA((N_Z - 1, Q)),
            pltpu.SemaphoreType.DMA((4,)),
            pltpu.SemaphoreType.REGULAR,
        ],
        compiler_params=pltpu.CompilerParams(
            collective_id=0,
            vmem_limit_bytes=64 * 1024 * 1024,
        ),
    )(O2, Wo)


# device time: 586510 ns/iter; 1.0748x vs baseline; 1.0294x over previous
import jax
import jax.numpy as jnp
from jax import lax
from jax.experimental import pallas as pl
from jax.experimental.pallas import tpu as pltpu

N_Z = 4
Q = 4


def kernel(O, Wo):
    B, S, Hs, D = O.shape
    K = Hs * D
    N = Wo.shape[1]
    S_out = S // N_Z
    RQ = S_out // Q

    OT = jnp.swapaxes(O.reshape(B, S, K), 1, 2)

    def body(o_hbm, w_hbm, out_hbm, w_vmem, o_stage, comm_ref,
             send_sems, recv_sems, local_sems, credit_sem):
        my_x = lax.axis_index("x")
        my_y = lax.axis_index("y")
        my_z = lax.axis_index("z")
        left = (my_z - 1) % N_Z
        right = (my_z + 1) % N_Z

        barrier_sem = pltpu.get_barrier_semaphore()
        for nbr in (left, right):
            pl.semaphore_signal(
                barrier_sem, inc=1,
                device_id=(my_x, my_y, nbr),
                device_id_type=pl.DeviceIdType.MESH,
            )
        pl.semaphore_wait(barrier_sem, 2)

        def stage_o(c, buf):
            cp = pltpu.make_async_copy(
                o_hbm.at[:, :, pl.ds(c * S_out, S_out)],
                o_stage.at[buf],
                local_sems.at[1 + buf],
            )
            cp.start()
            return cp

        def accum_block(slot, q, buf, init):
            for b in range(B):
                p = lax.dot_general(
                    o_stage[buf, b, :, q * RQ:(q + 1) * RQ],
                    w_vmem[...],
                    (((0,), (0,)), ((), ())),
                    preferred_element_type=jnp.float32,
                )
                if init:
                    comm_ref[slot, q, b, :, :] = p
                else:
                    comm_ref[slot, q, b, :, :] = comm_ref[slot, q, b, :, :] + p

        def make_rdma(t, q, s_slot, r_slot):
            return pltpu.make_async_remote_copy(
                src_ref=comm_ref.at[s_slot, q],
                dst_ref=comm_ref.at[r_slot, q],
                send_sem=send_sems.at[t, q],
                recv_sem=recv_sems.at[t, q],
                device_id=(my_x, my_y, right),
                device_id_type=pl.DeviceIdType.MESH,
            )

        wo_cp = pltpu.make_async_copy(w_hbm, w_vmem, local_sems.at[0])
        wo_cp.start()
        o_cp0 = stage_o((my_z - 1) % N_Z, 0)
        wo_cp.wait()
        o_cp0.wait()

        rdmas = {}

        for q in range(Q):
            accum_block(0, q, 0, init=True)
            r = make_rdma(0, q, 0, 1)
            r.start()
            rdmas[(0, q)] = r
        o_cp1 = stage_o((my_z - 2) % N_Z, 1)

        for t in range(1, N_Z - 1):
            s_slot = t % 2
            r_slot = (t + 1) % 2
            o_buf = t % 2
            o_cp = o_cp1 if o_buf == 1 else o_cp0
            o_cp.wait()
            for q in range(Q):
                rdmas[(t - 1, q)].wait_recv()
                accum_block(s_slot, q, o_buf, init=False)
            nxt = stage_o((my_z - 2 - t) % N_Z, (t + 1) % 2)
            if (t + 1) % 2 == 1:
                o_cp1 = nxt
            else:
                o_cp0 = nxt
            for q in range(Q):
                rdmas[(t - 1, q)].wait_send()
            pl.semaphore_signal(
                credit_sem, inc=1,
                device_id=(my_x, my_y, left),
                device_id_type=pl.DeviceIdType.MESH,
            )
            pl.semaphore_wait(credit_sem, 1)
            for q in range(Q):
                r = make_rdma(t, q, s_slot, r_slot)
                r.start()
                rdmas[(t, q)] = r

        o_cp1.wait()
        out_cps = []
        for q in range(Q):
            rdmas[(N_Z - 2, q)].wait_recv()
            accum_block(1, q, 1, init=False)
            for b in range(B):
                cp = pltpu.make_async_copy(
                    comm_ref.at[1, q, b],
                    out_hbm.at[b, pl.ds(q * RQ, RQ), :],
                    local_sems.at[3],
                )
                cp.start()
                out_cps.append(cp)
        for cp in out_cps:
            cp.wait()
        for q in range(Q):
            rdmas[(N_Z - 2, q)].wait_send()

    return pl.pallas_call(
        body,
        out_shape=jax.ShapeDtypeStruct((B, S_out, N), jnp.float32),
        in_specs=[
            pl.BlockSpec(memory_space=pltpu.MemorySpace.HBM),
            pl.BlockSpec(memory_space=pltpu.MemorySpace.HBM),
        ],
        out_specs=pl.BlockSpec(memory_space=pltpu.MemorySpace.HBM),
        scratch_shapes=[
            pltpu.VMEM((K, N), jnp.float32),
            pltpu.VMEM((2, B, K, S_out), jnp.float32),
            pltpu.VMEM((2, Q, B, RQ, N), jnp.float32),
            pltpu.SemaphoreType.DMA((N_Z - 1, Q)),
            pltpu.SemaphoreType.DMA((N_Z - 1, Q)),
            pltpu.SemaphoreType.DMA((4,)),
            pltpu.SemaphoreType.REGULAR,
        ],
        compiler_params=pltpu.CompilerParams(
            collective_id=0,
            vmem_limit_bytes=64 * 1024 * 1024,
        ),
    )(OT, Wo)


# device time: 572279 ns/iter; 1.1015x vs baseline; 1.0249x over previous
import jax
import jax.numpy as jnp
from jax import lax
from jax.experimental import pallas as pl
from jax.experimental.pallas import tpu as pltpu

N_Z = 4
Q = 4


def kernel(O, Wo):
    B, S, Hs, D = O.shape
    K = Hs * D
    N = Wo.shape[1]
    S_out = S // N_Z
    RQ = S_out // Q

    OT = jnp.swapaxes(O.reshape(B, S, K), 1, 2)

    def body(o_hbm, w_hbm, out_hbm, w_vmem, o_stage, comm_ref,
             send_sems, recv_sems, local_sems, credit_sem):
        my_x = lax.axis_index("x")
        my_y = lax.axis_index("y")
        my_z = lax.axis_index("z")
        left = (my_z - 1) % N_Z
        right = (my_z + 1) % N_Z

        def stage_o(c, buf):
            cp = pltpu.make_async_copy(
                o_hbm.at[:, :, pl.ds(c * S_out, S_out)],
                o_stage.at[buf],
                local_sems.at[1 + buf],
            )
            cp.start()
            return cp

        def accum_block(slot, q, buf, init):
            for b in range(B):
                p = lax.dot_general(
                    o_stage[buf, b, :, q * RQ:(q + 1) * RQ],
                    w_vmem[...],
                    (((0,), (0,)), ((), ())),
                    preferred_element_type=jnp.float32,
                )
                if init:
                    comm_ref[slot, q, b, :, :] = p
                else:
                    comm_ref[slot, q, b, :, :] = comm_ref[slot, q, b, :, :] + p

        def make_rdma(t, q, s_slot, r_slot):
            return pltpu.make_async_remote_copy(
                src_ref=comm_ref.at[s_slot, q],
                dst_ref=comm_ref.at[r_slot, q],
                send_sem=send_sems.at[t, q],
                recv_sem=recv_sems.at[t, q],
                device_id=(my_x, my_y, right),
                device_id_type=pl.DeviceIdType.MESH,
            )

        K2 = K // 2
        wo_a = pltpu.make_async_copy(
            w_hbm.at[0:K2], w_vmem.at[0:K2], local_sems.at[0])
        wo_a.start()
        wo_b = pltpu.make_async_copy(
            w_hbm.at[K2:K], w_vmem.at[K2:K], local_sems.at[4])
        wo_b.start()
        o_cp0 = stage_o((my_z - 1) % N_Z, 0)

        barrier_sem = pltpu.get_barrier_semaphore()
        for nbr in (left, right):
            pl.semaphore_signal(
                barrier_sem, inc=1,
                device_id=(my_x, my_y, nbr),
                device_id_type=pl.DeviceIdType.MESH,
            )
        pl.semaphore_wait(barrier_sem, 2)

        rdmas = {}

        o_cp0.wait()
        wo_a.wait()
        for b in range(B):
            comm_ref[0, 0, b, :, :] = lax.dot_general(
                o_stage[0, b, 0:K2, 0:RQ],
                w_vmem[0:K2, :],
                (((0,), (0,)), ((), ())),
                preferred_element_type=jnp.float32,
            )
        wo_b.wait()
        for b in range(B):
            comm_ref[0, 0, b, :, :] = comm_ref[0, 0, b, :, :] + lax.dot_general(
                o_stage[0, b, K2:K, 0:RQ],
                w_vmem[K2:K, :],
                (((0,), (0,)), ((), ())),
                preferred_element_type=jnp.float32,
            )
        r = make_rdma(0, 0, 0, 1)
        r.start()
        rdmas[(0, 0)] = r
        for q in range(1, Q):
            accum_block(0, q, 0, init=True)
            r = make_rdma(0, q, 0, 1)
            r.start()
            rdmas[(0, q)] = r
        o_cp1 = stage_o((my_z - 2) % N_Z, 1)

        for t in range(1, N_Z - 1):
            s_slot = t % 2
            r_slot = (t + 1) % 2
            o_buf = t % 2
            o_cp = o_cp1 if o_buf == 1 else o_cp0
            o_cp.wait()
            nxt = stage_o((my_z - 2 - t) % N_Z, (t + 1) % 2)
            if (t + 1) % 2 == 1:
                o_cp1 = nxt
            else:
                o_cp0 = nxt
            for q in range(Q):
                rdmas[(t - 1, q)].wait_recv()
                accum_block(s_slot, q, o_buf, init=False)
                rdmas[(t - 1, q)].wait_send()
                pl.semaphore_signal(
                    credit_sem, inc=1,
                    device_id=(my_x, my_y, left),
                    device_id_type=pl.DeviceIdType.MESH,
                )
                pl.semaphore_wait(credit_sem, 1)
                r = make_rdma(t, q, s_slot, r_slot)
                r.start()
                rdmas[(t, q)] = r

        o_cp1.wait()
        out_cps = []
        for q in range(Q):
            rdmas[(N_Z - 2, q)].wait_recv()
            accum_block(1, q, 1, init=False)
            for b in range(B):
                cp = pltpu.make_async_copy(
                    comm_ref.at[1, q, b],
                    out_hbm.at[b, pl.ds(q * RQ, RQ), :],
                    local_sems.at[3],
                )
                cp.start()
                out_cps.append(cp)
        for cp in out_cps:
            cp.wait()
        for q in range(Q):
            rdmas[(N_Z - 2, q)].wait_send()

    return pl.pallas_call(
        body,
        out_shape=jax.ShapeDtypeStruct((B, S_out, N), jnp.float32),
        in_specs=[
            pl.BlockSpec(memory_space=pltpu.MemorySpace.HBM),
            pl.BlockSpec(memory_space=pltpu.MemorySpace.HBM),
        ],
        out_specs=pl.BlockSpec(memory_space=pltpu.MemorySpace.HBM),
        scratch_shapes=[
            pltpu.VMEM((K, N), jnp.float32),
            pltpu.VMEM((2, B, K, S_out), jnp.float32),
            pltpu.VMEM((2, Q, B, RQ, N), jnp.float32),
            pltpu.SemaphoreType.DMA((N_Z - 1, Q)),
            pltpu.SemaphoreType.DMA((N_Z - 1, Q)),
            pltpu.SemaphoreType.DMA((5,)),
            pltpu.SemaphoreType.REGULAR,
        ],
        compiler_params=pltpu.CompilerParams(
            collective_id=0,
            vmem_limit_bytes=64 * 1024 * 1024,
        ),
    )(OT, Wo)


# device time: 571100 ns/iter; 1.1038x vs baseline; 1.0021x over previous
import jax
import jax.numpy as jnp
from jax import lax
from jax.experimental import pallas as pl
from jax.experimental.pallas import tpu as pltpu

N_Z = 4
Q = 4


def kernel(O, Wo):
    B, S, Hs, D = O.shape
    K = Hs * D
    N = Wo.shape[1]
    S_out = S // N_Z
    RQ = S_out // Q

    OT = jnp.swapaxes(O.reshape(B, S, K), 1, 2)

    def body(o_hbm, w_hbm, out_hbm, w_vmem, o_stage, comm_ref,
             send_sems, recv_sems, local_sems, credit_sem):
        my_x = lax.axis_index("x")
        my_y = lax.axis_index("y")
        my_z = lax.axis_index("z")
        left = (my_z - 1) % N_Z
        right = (my_z + 1) % N_Z

        def stage_o(c, buf):
            cp = pltpu.make_async_copy(
                o_hbm.at[:, :, pl.ds(c * S_out, S_out)],
                o_stage.at[buf],
                local_sems.at[1 + buf],
            )
            cp.start()
            return cp

        def accum_block(slot, q, buf, init):
            for b in range(B):
                p = lax.dot_general(
                    o_stage[buf, b, :, q * RQ:(q + 1) * RQ],
                    w_vmem[...],
                    (((0,), (0,)), ((), ())),
                    preferred_element_type=jnp.float32,
                )
                if init:
                    comm_ref[slot, q, b, :, :] = p
                else:
                    comm_ref[slot, q, b, :, :] = comm_ref[slot, q, b, :, :] + p

        def make_rdma(t, q, s_slot, r_slot):
            return pltpu.make_async_remote_copy(
                src_ref=comm_ref.at[s_slot, q],
                dst_ref=comm_ref.at[r_slot, q],
                send_sem=send_sems.at[t, q],
                recv_sem=recv_sems.at[t, q],
                device_id=(my_x, my_y, right),
                device_id_type=pl.DeviceIdType.MESH,
            )

        K2 = K // 2
        wo_a = pltpu.make_async_copy(
            w_hbm.at[0:K2], w_vmem.at[0:K2], local_sems.at[0])
        wo_a.start()
        wo_b = pltpu.make_async_copy(
            w_hbm.at[K2:K], w_vmem.at[K2:K], local_sems.at[4])
        wo_b.start()
        o_cp0 = stage_o((my_z - 1) % N_Z, 0)

        barrier_sem = pltpu.get_barrier_semaphore()
        for nbr in (left, right):
            pl.semaphore_signal(
                barrier_sem, inc=1,
                device_id=(my_x, my_y, nbr),
                device_id_type=pl.DeviceIdType.MESH,
            )
        pl.semaphore_wait(barrier_sem, 2)

        rdmas = {}

        N2 = N // 2
        o_cp0.wait()
        wo_a.wait()
        for b in range(B):
            comm_ref[0, 0, b, :, 0:N2] = lax.dot_general(
                o_stage[0, b, 0:K2, 0:RQ],
                w_vmem[0:K2, 0:N2],
                (((0,), (0,)), ((), ())),
                preferred_element_type=jnp.float32,
            )
        wo_b.wait()
        for b in range(B):
            comm_ref[0, 0, b, :, 0:N2] = (
                comm_ref[0, 0, b, :, 0:N2] + lax.dot_general(
                    o_stage[0, b, K2:K, 0:RQ],
                    w_vmem[K2:K, 0:N2],
                    (((0,), (0,)), ((), ())),
                    preferred_element_type=jnp.float32,
                )
            )
        r0a = pltpu.make_async_remote_copy(
            src_ref=comm_ref.at[0, 0, :, :, 0:N2],
            dst_ref=comm_ref.at[1, 0, :, :, 0:N2],
            send_sem=send_sems.at[0, 0],
            recv_sem=recv_sems.at[0, 0],
            device_id=(my_x, my_y, right),
            device_id_type=pl.DeviceIdType.MESH,
        )
        r0a.start()
        for b in range(B):
            comm_ref[0, 0, b, :, N2:N] = lax.dot_general(
                o_stage[0, b, :, 0:RQ],
                w_vmem[:, N2:N],
                (((0,), (0,)), ((), ())),
                preferred_element_type=jnp.float32,
            )
        r0b = pltpu.make_async_remote_copy(
            src_ref=comm_ref.at[0, 0, :, :, N2:N],
            dst_ref=comm_ref.at[1, 0, :, :, N2:N],
            send_sem=send_sems.at[0, Q],
            recv_sem=recv_sems.at[0, Q],
            device_id=(my_x, my_y, right),
            device_id_type=pl.DeviceIdType.MESH,
        )
        r0b.start()
        rdmas[(0, 0)] = [r0a, r0b]
        for q in range(1, Q):
            accum_block(0, q, 0, init=True)
            r = make_rdma(0, q, 0, 1)
            r.start()
            rdmas[(0, q)] = [r]
        o_cp1 = stage_o((my_z - 2) % N_Z, 1)

        for t in range(1, N_Z - 1):
            s_slot = t % 2
            r_slot = (t + 1) % 2
            o_buf = t % 2
            o_cp = o_cp1 if o_buf == 1 else o_cp0
            o_cp.wait()
            nxt = stage_o((my_z - 2 - t) % N_Z, (t + 1) % 2)
            if (t + 1) % 2 == 1:
                o_cp1 = nxt
            else:
                o_cp0 = nxt
            for q in range(Q):
                for rr in rdmas[(t - 1, q)]:
                    rr.wait_recv()
                accum_block(s_slot, q, o_buf, init=False)
                for rr in rdmas[(t - 1, q)]:
                    rr.wait_send()
                pl.semaphore_signal(
                    credit_sem, inc=1,
                    device_id=(my_x, my_y, left),
                    device_id_type=pl.DeviceIdType.MESH,
                )
                pl.semaphore_wait(credit_sem, 1)
                r = make_rdma(t, q, s_slot, r_slot)
                r.start()
                rdmas[(t, q)] = [r]

        o_cp1.wait()
        out_cps = []
        for q in range(Q):
            for rr in rdmas[(N_Z - 2, q)]:
                rr.wait_recv()
            accum_block(1, q, 1, init=False)
            for b in range(B):
                cp = pltpu.make_async_copy(
                    comm_ref.at[1, q, b],
                    out_hbm.at[b, pl.ds(q * RQ, RQ), :],
                    local_sems.at[3],
                )
                cp.start()
                out_cps.append(cp)
        for cp in out_cps:
            cp.wait()
        for q in range(Q):
            for rr in rdmas[(N_Z - 2, q)]:
                rr.wait_send()

    return pl.pallas_call(
        body,
        out_shape=jax.ShapeDtypeStruct((B, S_out, N), jnp.float32),
        in_specs=[
            pl.BlockSpec(memory_space=pltpu.MemorySpace.HBM),
            pl.BlockSpec(memory_space=pltpu.MemorySpace.HBM),
        ],
        out_specs=pl.BlockSpec(memory_space=pltpu.MemorySpace.HBM),
        scratch_shapes=[
            pltpu.VMEM((K, N), jnp.float32),
            pltpu.VMEM((2, B, K, S_out), jnp.float32),
            pltpu.VMEM((2, Q, B, RQ, N), jnp.float32),
            pltpu.SemaphoreType.DMA((N_Z - 1, Q + 1)),
            pltpu.SemaphoreType.DMA((N_Z - 1, Q + 1)),
            pltpu.SemaphoreType.DMA((5,)),
            pltpu.SemaphoreType.REGULAR,
        ],
        compiler_params=pltpu.CompilerParams(
            collective_id=0,
            vmem_limit_bytes=64 * 1024 * 1024,
        ),
    )(OT, Wo)
